# baseline (speedup 1.0000x reference)
import functools

import jax
import jax.numpy as jnp
from jax import lax
from jax.experimental import pallas as pl
from jax.experimental.pallas import tpu as pltpu
from jax.experimental.pallas import tpu_sc as plsc

B, S = 16384, 200
NC, NS, LANES = 2, 16, 16
NW = NC * NS
NUNITS = S * 4
PER_W = NUNITS // NW
ROWS = 32
CSTEPS = 128 // LANES

_mesh = plsc.VectorSubcoreMesh(core_axis_name="c", subcore_axis_name="s")


@functools.partial(
    pl.kernel,
    out_type=jax.ShapeDtypeStruct((S, 128, 2, 128), jnp.float32),
    mesh=_mesh,
    scratch_types=[
        pltpu.VMEM((128,), jnp.float32),
        pltpu.VMEM((ROWS, 128), jnp.int32),
        pltpu.VMEM((ROWS, 128), jnp.int32),
        pltpu.VMEM((ROWS, 2, 128), jnp.float32),
        pltpu.VMEM((ROWS, 2, 128), jnp.float32),
        pltpu.SemaphoreType.DMA,
        pltpu.SemaphoreType.DMA,
        pltpu.SemaphoreType.DMA,
        pltpu.SemaphoreType.DMA,
    ],
    compiler_params=pltpu.CompilerParams(needs_layout_passes=False),
)
def _emb_lookup(idx_hbm, tab_hbm, out_hbm, tab_v,
                idx_a, idx_b, out_a, out_b, sia, sib, soa, sob):
    wid = lax.axis_index("s") * NC + lax.axis_index("c")
    base = wid * PER_W

    idx_bufs = (idx_a, idx_b)
    out_bufs = (out_a, out_b)
    in_sems = (sia, sib)
    out_sems = (soa, sob)

    def in_copy(t):
        u = base + t
        j, q = u // 4, u % 4
        return pltpu.async_copy(
            idx_hbm.at[j // 8, pl.ds(q * ROWS, ROWS), j % 8],
            idx_bufs[t % 2], in_sems[t % 2])

    def out_copy(t):
        u = base + t
        j, q = u // 4, u % 4
        return pltpu.async_copy(
            out_bufs[t % 2],
            out_hbm.at[j, pl.ds(q * ROWS, ROWS)],
            out_sems[t % 2])

    pending_in = in_copy(0)
    pltpu.sync_copy(tab_hbm, tab_v)
    pending_out = [None, None]
    for t in range(PER_W):
        nxt = in_copy(t + 1) if t + 1 < PER_W else None
        pending_in.wait()
        pending_in = nxt
        if pending_out[t % 2] is not None:
            pending_out[t % 2].wait()
        iv_ref, ov_ref = idx_bufs[t % 2], out_bufs[t % 2]

        @plsc.parallel_loop(0, ROWS * CSTEPS, step=1, unroll=16)
        def step(i):
            r = i // CSTEPS
            c = (i % CSTEPS) * LANES
            iv = iv_ref[r, pl.ds(c, LANES)]
            g = iv * 2
            ov_ref[r, 0, pl.ds(c, LANES)] = plsc.load_gather(tab_v, [g])
            ov_ref[r, 1, pl.ds(c, LANES)] = plsc.load_gather(tab_v, [g + 1])

        pending_out[t % 2] = out_copy(t)
    pending_out[0].wait()
    pending_out[1].wait()


def kernel(indices, C):
    tab = jnp.zeros((128,), jnp.float32).at[:72].set(C.reshape(-1))
    xr = indices.T.reshape(S // 8, 8, 128, 128).transpose(0, 2, 1, 3)
    w = _emb_lookup(xr, tab)
    return w.transpose(1, 3, 0, 2).reshape(B, S, 2)
